# Initial kernel scaffold; baseline (speedup 1.0000x reference)
import jax
import jax.numpy as jnp
from jax.experimental import pallas as pl


def kernel(features, labels, directions):
    raise NotImplementedError("write your pallas kernel here")


# speedup vs baseline: 19.3090x; 19.3090x over previous
import jax
import jax.numpy as jnp
from jax import lax
from jax.experimental import pallas as pl
from jax.experimental.pallas import tpu as pltpu
from jax.experimental.pallas import tpu_sc as plsc

INV_TEMP = 10.0
N, C, H, W = 16, 512, 16, 16
P = H * W
NC, NS, L = 2, 16, 16
NW = NC * NS
PPS = P // NW


def _gram_body(x_ref, g_ref):
    x = x_ref[0]
    n2 = jnp.sum(x * x, axis=0, keepdims=True)
    xn = x / jnp.maximum(jnp.sqrt(n2), 1e-12)
    g = lax.dot_general(
        xn, xn, (((0,), (0,)), ((), ())),
        preferred_element_type=jnp.float32,
        precision=lax.Precision.HIGHEST,
    )
    g_ref[0] = g * INV_TEMP


def _gram(feats):
    return pl.pallas_call(
        _gram_body,
        grid=(N,),
        in_specs=[pl.BlockSpec((1, C, P), lambda b: (b, 0, 0))],
        out_specs=pl.BlockSpec((1, P, P), lambda b: (b, 0, 0)),
        out_shape=jax.ShapeDtypeStruct((N, P, P), jnp.float32),
    )(feats)


def _sc_body(gt_hbm, labt_hbm, d0_hbm, d1_hbm, denom_hbm, svs_hbm,
             rows_v, labrows_v, labc_v, d0v, d1v, outd, outs, sem):
    wid = lax.axis_index("s") * NC + lax.axis_index("c")
    base = wid * PPS
    pltpu.sync_copy(d0_hbm.at[pl.ds(base, PPS)], d0v)
    pltpu.sync_copy(d1_hbm.at[pl.ds(base, PPS)], d1v)
    pltpu.sync_copy(labt_hbm.at[pl.ds(base, PPS)], labc_v)
    for pi in range(PPS):
        p = base + pi
        i = p // W
        j = p - i * W
        ni = i + d0v[pi]
        nj = j + d1v[pi]
        valid = (ni >= 0) & (ni < H) & (nj >= 0) & (nj < W)
        q = jnp.clip(ni, 0, H - 1) * W + jnp.clip(nj, 0, W - 1)
        fz = jnp.zeros((L,), jnp.float32)
        v = jnp.where(valid, fz + 1.0, fz)
        pltpu.async_copy(gt_hbm.at[p * P + q], rows_v, sem).wait()
        pltpu.async_copy(labt_hbm.at[q], labrows_v, sem).wait()
        labc = labc_v[pi]
        acc_d = fz
        acc_s = fz
        for n in range(N):
            nsplat = jnp.clip(q * 0 + n, 0, L - 1)
            vn = jnp.take(v, nsplat)
            s = rows_v[n]
            labn = labrows_v[n]
            mask = jnp.where(labn == labc, fz + 1.0, fz)
            acc_d = acc_d + jnp.exp(s) * mask * vn
            acc_s = acc_s + vn * s
        outd[pi] = acc_d
        outs[pi] = acc_s
    pltpu.sync_copy(outd, denom_hbm.at[pl.ds(base, PPS)])
    pltpu.sync_copy(outs, svs_hbm.at[pl.ds(base, PPS)])


def _sc_gather(gt, labt, d0t, d1t):
    fn = pl.kernel(
        _sc_body,
        out_type=[jax.ShapeDtypeStruct((P, N), jnp.float32),
                  jax.ShapeDtypeStruct((P, N), jnp.float32)],
        mesh=plsc.VectorSubcoreMesh(core_axis_name="c", subcore_axis_name="s",
                                    num_cores=NC, num_subcores=NS),
        compiler_params=pltpu.CompilerParams(use_tc_tiling_on_sc=False),
        scratch_types=[
            pltpu.VMEM((L, N), jnp.float32),
            pltpu.VMEM((L, N), jnp.int32),
            pltpu.VMEM((PPS, N), jnp.int32),
            pltpu.VMEM((PPS, L), jnp.int32),
            pltpu.VMEM((PPS, L), jnp.int32),
            pltpu.VMEM((PPS, L), jnp.float32),
            pltpu.VMEM((PPS, L), jnp.float32),
            pltpu.SemaphoreType.DMA,
        ],
    )
    return fn(gt, labt, d0t, d1t)


def _loss_body(den_ref, svs_ref, d0_ref, d1_ref, out_ref):
    den = den_ref[...]
    svs = svs_ref[...]
    d0 = d0_ref[...]
    d1 = d1_ref[...]
    pidx = lax.broadcasted_iota(jnp.int32, (P, L), 0)
    i = pidx // W
    j = pidx - i * W
    ni = i + d0
    nj = j + d1
    valid = (ni >= 0) & (ni < H) & (nj >= 0) & (nj < W)
    v = valid.astype(jnp.float32)
    count = jnp.sum(v, axis=1)
    tot = count * jnp.sum(jnp.log(den + 1e-6), axis=1) - jnp.sum(svs, axis=1)
    loss_p = jnp.where(count > 0, tot / (N * jnp.maximum(count, 1.0)), 0.0)
    out_ref[0, 0] = jnp.sum(loss_p) / P


def _loss(denom, svs, d0t, d1t):
    r = pl.pallas_call(
        _loss_body,
        in_specs=[
            pl.BlockSpec((P, N), lambda: (0, 0)),
            pl.BlockSpec((P, N), lambda: (0, 0)),
            pl.BlockSpec((P, L), lambda: (0, 0)),
            pl.BlockSpec((P, L), lambda: (0, 0)),
        ],
        out_specs=pl.BlockSpec(memory_space=pltpu.SMEM),
        out_shape=jax.ShapeDtypeStruct((1, 1), jnp.float32),
    )(denom, svs, d0t, d1t)
    return r[0, 0]


def kernel(features, labels, directions):
    feats = features.reshape(N, C, P)
    labt = labels.reshape(N, P).T
    d0t = directions[:, 0].reshape(N, P).T
    d1t = directions[:, 1].reshape(N, P).T
    g = _gram(feats)
    gt = jnp.transpose(g, (1, 2, 0)).reshape(P * P, N)
    denom, svs = _sc_gather(gt, labt, d0t, d1t)
    return _loss(denom, svs, d0t, d1t)


# speedup vs baseline: 20.0940x; 1.0407x over previous
import jax
import jax.numpy as jnp
from jax import lax
from jax.experimental import pallas as pl
from jax.experimental.pallas import tpu as pltpu
from jax.experimental.pallas import tpu_sc as plsc

INV_TEMP = 10.0
N, C, H, W = 16, 512, 16, 16
P = H * W
NC, NS, L = 2, 16, 16
NW = NC * NS
PPS = P // NW


def _gram_body(x_ref, g_ref):
    x = x_ref[0]
    n2 = jnp.sum(x * x, axis=0, keepdims=True)
    xn = x / jnp.maximum(jnp.sqrt(n2), 1e-12)
    g = lax.dot_general(
        xn, xn, (((0,), (0,)), ((), ())),
        preferred_element_type=jnp.float32,
    )
    g_ref[0] = g * INV_TEMP


def _gram(feats):
    return pl.pallas_call(
        _gram_body,
        grid=(N,),
        in_specs=[pl.BlockSpec((1, C, P), lambda b: (b, 0, 0))],
        out_specs=pl.BlockSpec((1, P, P), lambda b: (b, 0, 0)),
        out_shape=jax.ShapeDtypeStruct((N, P, P), jnp.float32),
    )(feats)


def _sc_body(gt_hbm, labt_hbm, d0_hbm, d1_hbm, denom_hbm, svs_hbm,
             rows_v, labrows_v, labc_v, d0v, d1v, outd, outs, sem):
    wid = lax.axis_index("s") * NC + lax.axis_index("c")
    base = wid * PPS
    pltpu.sync_copy(d0_hbm.at[pl.ds(base, PPS)], d0v)
    pltpu.sync_copy(d1_hbm.at[pl.ds(base, PPS)], d1v)
    pltpu.sync_copy(labt_hbm.at[pl.ds(base, PPS)], labc_v)
    for pi in range(PPS):
        p = base + pi
        i = p // W
        j = p - i * W
        ni = i + d0v[pi]
        nj = j + d1v[pi]
        valid = (ni >= 0) & (ni < H) & (nj >= 0) & (nj < W)
        q = jnp.clip(ni, 0, H - 1) * W + jnp.clip(nj, 0, W - 1)
        fz = jnp.zeros((L,), jnp.float32)
        v = jnp.where(valid, fz + 1.0, fz)
        pltpu.async_copy(gt_hbm.at[p * P + q], rows_v, sem).wait()
        pltpu.async_copy(labt_hbm.at[q], labrows_v, sem).wait()
        labc = labc_v[pi]
        acc_d = fz
        acc_s = fz
        for n in range(N):
            nsplat = jnp.clip(q * 0 + n, 0, L - 1)
            vn = jnp.take(v, nsplat)
            s = rows_v[n]
            labn = labrows_v[n]
            mask = jnp.where(labn == labc, fz + 1.0, fz)
            acc_d = acc_d + jnp.exp(s) * mask * vn
            acc_s = acc_s + vn * s
        outd[pi] = acc_d
        outs[pi] = acc_s
    pltpu.sync_copy(outd, denom_hbm.at[pl.ds(base, PPS)])
    pltpu.sync_copy(outs, svs_hbm.at[pl.ds(base, PPS)])


def _sc_gather(gt, labt, d0t, d1t):
    fn = pl.kernel(
        _sc_body,
        out_type=[jax.ShapeDtypeStruct((P, N), jnp.float32),
                  jax.ShapeDtypeStruct((P, N), jnp.float32)],
        mesh=plsc.VectorSubcoreMesh(core_axis_name="c", subcore_axis_name="s",
                                    num_cores=NC, num_subcores=NS),
        compiler_params=pltpu.CompilerParams(use_tc_tiling_on_sc=False),
        scratch_types=[
            pltpu.VMEM((L, N), jnp.float32),
            pltpu.VMEM((L, N), jnp.int32),
            pltpu.VMEM((PPS, N), jnp.int32),
            pltpu.VMEM((PPS, L), jnp.int32),
            pltpu.VMEM((PPS, L), jnp.int32),
            pltpu.VMEM((PPS, L), jnp.float32),
            pltpu.VMEM((PPS, L), jnp.float32),
            pltpu.SemaphoreType.DMA,
        ],
    )
    return fn(gt, labt, d0t, d1t)


def _loss_body(den_ref, svs_ref, d0_ref, d1_ref, out_ref):
    den = den_ref[...]
    svs = svs_ref[...]
    d0 = d0_ref[...]
    d1 = d1_ref[...]
    pidx = lax.broadcasted_iota(jnp.int32, (P, L), 0)
    i = pidx // W
    j = pidx - i * W
    ni = i + d0
    nj = j + d1
    valid = (ni >= 0) & (ni < H) & (nj >= 0) & (nj < W)
    v = valid.astype(jnp.float32)
    count = jnp.sum(v, axis=1)
    tot = count * jnp.sum(jnp.log(den + 1e-6), axis=1) - jnp.sum(svs, axis=1)
    loss_p = jnp.where(count > 0, tot / (N * jnp.maximum(count, 1.0)), 0.0)
    out_ref[0, 0] = jnp.sum(loss_p) / P


def _loss(denom, svs, d0t, d1t):
    r = pl.pallas_call(
        _loss_body,
        in_specs=[
            pl.BlockSpec((P, N), lambda: (0, 0)),
            pl.BlockSpec((P, N), lambda: (0, 0)),
            pl.BlockSpec((P, L), lambda: (0, 0)),
            pl.BlockSpec((P, L), lambda: (0, 0)),
        ],
        out_specs=pl.BlockSpec(memory_space=pltpu.SMEM),
        out_shape=jax.ShapeDtypeStruct((1, 1), jnp.float32),
    )(denom, svs, d0t, d1t)
    return r[0, 0]


def kernel(features, labels, directions):
    feats = features.reshape(N, C, P)
    labt = labels.reshape(N, P).T
    d0t = directions[:, 0].reshape(N, P).T
    d1t = directions[:, 1].reshape(N, P).T
    g = _gram(feats)
    gt = jnp.transpose(g, (1, 2, 0)).reshape(P * P, N)
    denom, svs = _sc_gather(gt, labt, d0t, d1t)
    return _loss(denom, svs, d0t, d1t)


# speedup vs baseline: 21.6456x; 1.0772x over previous
import jax
import jax.numpy as jnp
from jax import lax
from jax.experimental import pallas as pl
from jax.experimental.pallas import tpu as pltpu
from jax.experimental.pallas import tpu_sc as plsc

INV_TEMP = 10.0
N, C, H, W = 16, 512, 16, 16
P = H * W
NC, NS, L = 2, 16, 16
NW = NC * NS
PPS = P // NW


def _gram_body(x_ref, g_ref):
    x = x_ref[0]
    n2 = jnp.sum(x * x, axis=0, keepdims=True)
    xn = x / jnp.maximum(jnp.sqrt(n2), 1e-12)
    g = lax.dot_general(
        xn, xn, (((0,), (0,)), ((), ())),
        preferred_element_type=jnp.float32,
    )
    g_ref[0] = g * INV_TEMP


def _gram(feats):
    return pl.pallas_call(
        _gram_body,
        grid=(N,),
        in_specs=[pl.BlockSpec((1, C, P), lambda b: (b, 0, 0))],
        out_specs=pl.BlockSpec((1, P, P), lambda b: (b, 0, 0)),
        out_shape=jax.ShapeDtypeStruct((N, P, P), jnp.float32),
    )(feats)


def _sc_body(gt_hbm, labt_hbm, d0_hbm, d1_hbm, denom_hbm, svs_hbm,
             rows_v, labrows_v, labc_v, d0v, d1v, vv, idxg, idxl,
             outd, outs, semg, seml):
    wid = lax.axis_index("s") * NC + lax.axis_index("c")
    base = wid * PPS
    pltpu.sync_copy(d0_hbm.at[pl.ds(base, PPS)], d0v)
    pltpu.sync_copy(d1_hbm.at[pl.ds(base, PPS)], d1v)
    pltpu.sync_copy(labt_hbm.at[pl.ds(base, PPS)], labc_v)
    for pi in range(PPS):
        p = base + pi
        i = p // W
        j = p - i * W
        ni = i + d0v[pi]
        nj = j + d1v[pi]
        valid = (ni >= 0) & (ni < H) & (nj >= 0) & (nj < W)
        q = jnp.clip(ni, 0, H - 1) * W + jnp.clip(nj, 0, W - 1)
        fz = jnp.zeros((L,), jnp.float32)
        vv[pi] = jnp.where(valid, fz + 1.0, fz)
        idxg[pl.ds(pi * L, L)] = p * P + q
        idxl[pl.ds(pi * L, L)] = q
    cg = pltpu.async_copy(gt_hbm.at[idxg], rows_v, semg)
    cl = pltpu.async_copy(labt_hbm.at[idxl], labrows_v, seml)
    cg.wait()
    cl.wait()
    for pi in range(PPS):
        v = vv[pi]
        labc = labc_v[pi]
        fz = jnp.zeros((L,), jnp.float32)
        acc_d = fz
        acc_s = fz
        for n in range(N):
            r = pi * L + n
            nsplat = jnp.clip(d0v[pi] * 0 + n, 0, L - 1)
            vn = jnp.take(v, nsplat)
            s = rows_v[r]
            labn = labrows_v[r]
            mask = jnp.where(labn == labc, fz + 1.0, fz)
            acc_d = acc_d + jnp.exp(s) * mask * vn
            acc_s = acc_s + vn * s
        outd[pi] = acc_d
        outs[pi] = acc_s
    pltpu.sync_copy(outd, denom_hbm.at[pl.ds(base, PPS)])
    pltpu.sync_copy(outs, svs_hbm.at[pl.ds(base, PPS)])


def _sc_gather(gt, labt, d0t, d1t):
    fn = pl.kernel(
        _sc_body,
        out_type=[jax.ShapeDtypeStruct((P, N), jnp.float32),
                  jax.ShapeDtypeStruct((P, N), jnp.float32)],
        mesh=plsc.VectorSubcoreMesh(core_axis_name="c", subcore_axis_name="s",
                                    num_cores=NC, num_subcores=NS),
        compiler_params=pltpu.CompilerParams(use_tc_tiling_on_sc=False),
        scratch_types=[
            pltpu.VMEM((PPS * L, N), jnp.float32),
            pltpu.VMEM((PPS * L, N), jnp.int32),
            pltpu.VMEM((PPS, N), jnp.int32),
            pltpu.VMEM((PPS, L), jnp.int32),
            pltpu.VMEM((PPS, L), jnp.int32),
            pltpu.VMEM((PPS, L), jnp.float32),
            pltpu.VMEM((PPS * L,), jnp.int32),
            pltpu.VMEM((PPS * L,), jnp.int32),
            pltpu.VMEM((PPS, L), jnp.float32),
            pltpu.VMEM((PPS, L), jnp.float32),
            pltpu.SemaphoreType.DMA,
            pltpu.SemaphoreType.DMA,
        ],
    )
    return fn(gt, labt, d0t, d1t)


def _loss_body(den_ref, svs_ref, d0_ref, d1_ref, out_ref):
    den = den_ref[...]
    svs = svs_ref[...]
    d0 = d0_ref[...]
    d1 = d1_ref[...]
    pidx = lax.broadcasted_iota(jnp.int32, (P, L), 0)
    i = pidx // W
    j = pidx - i * W
    ni = i + d0
    nj = j + d1
    valid = (ni >= 0) & (ni < H) & (nj >= 0) & (nj < W)
    v = valid.astype(jnp.float32)
    count = jnp.sum(v, axis=1)
    tot = count * jnp.sum(jnp.log(den + 1e-6), axis=1) - jnp.sum(svs, axis=1)
    loss_p = jnp.where(count > 0, tot / (N * jnp.maximum(count, 1.0)), 0.0)
    out_ref[0, 0] = jnp.sum(loss_p) / P


def _loss(denom, svs, d0t, d1t):
    r = pl.pallas_call(
        _loss_body,
        in_specs=[
            pl.BlockSpec((P, N), lambda: (0, 0)),
            pl.BlockSpec((P, N), lambda: (0, 0)),
            pl.BlockSpec((P, L), lambda: (0, 0)),
            pl.BlockSpec((P, L), lambda: (0, 0)),
        ],
        out_specs=pl.BlockSpec(memory_space=pltpu.SMEM),
        out_shape=jax.ShapeDtypeStruct((1, 1), jnp.float32),
    )(denom, svs, d0t, d1t)
    return r[0, 0]


def kernel(features, labels, directions):
    feats = features.reshape(N, C, P)
    labt = labels.reshape(N, P).T
    d0t = directions[:, 0].reshape(N, P).T
    d1t = directions[:, 1].reshape(N, P).T
    g = _gram(feats)
    gt = jnp.transpose(g, (1, 2, 0)).reshape(P * P, N)
    denom, svs = _sc_gather(gt, labt, d0t, d1t)
    return _loss(denom, svs, d0t, d1t)


# speedup vs baseline: 21.7672x; 1.0056x over previous
import jax
import jax.numpy as jnp
from jax import lax
from jax.experimental import pallas as pl
from jax.experimental.pallas import tpu as pltpu
from jax.experimental.pallas import tpu_sc as plsc

INV_TEMP = 10.0
N, C, H, W = 16, 512, 16, 16
P = H * W
NC, NS, L = 1, 16, 16
NW = NC * NS
PPS = P // NW


def _gram_body(x_ref, g_ref):
    x = x_ref[0]
    n2 = jnp.sum(x * x, axis=0, keepdims=True)
    xn = x / jnp.maximum(jnp.sqrt(n2), 1e-12)
    g = lax.dot_general(
        xn, xn, (((0,), (0,)), ((), ())),
        preferred_element_type=jnp.float32,
    )
    g_ref[0] = g * INV_TEMP


def _gram(feats):
    return pl.pallas_call(
        _gram_body,
        grid=(N,),
        in_specs=[pl.BlockSpec((1, C, P), lambda b: (b, 0, 0))],
        out_specs=pl.BlockSpec((1, P, P), lambda b: (b, 0, 0)),
        out_shape=jax.ShapeDtypeStruct((N, P, P), jnp.float32),
    )(feats)


def _sc_body(gt_hbm, labt_hbm, d0_hbm, d1_hbm, denom_hbm, svs_hbm,
             rows_v, labrows_v, labc_v, d0v, d1v, vv, idxg, idxl,
             outd, outs, semg, seml):
    wid = lax.axis_index("s") * NC + lax.axis_index("c")
    base = wid * PPS
    pltpu.sync_copy(d0_hbm.at[pl.ds(base, PPS)], d0v)
    pltpu.sync_copy(d1_hbm.at[pl.ds(base, PPS)], d1v)
    pltpu.sync_copy(labt_hbm.at[pl.ds(base, PPS)], labc_v)
    for pi in range(PPS):
        p = base + pi
        i = p // W
        j = p - i * W
        ni = i + d0v[pi]
        nj = j + d1v[pi]
        valid = (ni >= 0) & (ni < H) & (nj >= 0) & (nj < W)
        q = jnp.clip(ni, 0, H - 1) * W + jnp.clip(nj, 0, W - 1)
        fz = jnp.zeros((L,), jnp.float32)
        vv[pi] = jnp.where(valid, fz + 1.0, fz)
        idxg[pl.ds(pi * L, L)] = p * P + q
        idxl[pl.ds(pi * L, L)] = q
    copies = []
    for ck in range(PPS * L // 128):
        d = pl.ds(ck * 128, 128)
        copies.append(pltpu.async_copy(gt_hbm.at[idxg.at[d]], rows_v.at[d], semg))
        copies.append(pltpu.async_copy(labt_hbm.at[idxl.at[d]], labrows_v.at[d], seml))
    for c in copies:
        c.wait()
    for pi in range(PPS):
        v = vv[pi]
        labc = labc_v[pi]
        fz = jnp.zeros((L,), jnp.float32)
        acc_d = fz
        acc_s = fz
        for n in range(N):
            r = pi * L + n
            nsplat = jnp.clip(d0v[pi] * 0 + n, 0, L - 1)
            vn = jnp.take(v, nsplat)
            s = rows_v[r]
            labn = labrows_v[r]
            mask = jnp.where(labn == labc, fz + 1.0, fz)
            acc_d = acc_d + jnp.exp(s) * mask * vn
            acc_s = acc_s + vn * s
        outd[pi] = acc_d
        outs[pi] = acc_s
    pltpu.sync_copy(outd, denom_hbm.at[pl.ds(base, PPS)])
    pltpu.sync_copy(outs, svs_hbm.at[pl.ds(base, PPS)])


def _sc_gather(gt, labt, d0t, d1t):
    fn = pl.kernel(
        _sc_body,
        out_type=[jax.ShapeDtypeStruct((P, N), jnp.float32),
                  jax.ShapeDtypeStruct((P, N), jnp.float32)],
        mesh=plsc.VectorSubcoreMesh(core_axis_name="c", subcore_axis_name="s",
                                    num_cores=NC, num_subcores=NS),
        compiler_params=pltpu.CompilerParams(use_tc_tiling_on_sc=False),
        scratch_types=[
            pltpu.VMEM((PPS * L, N), jnp.float32),
            pltpu.VMEM((PPS * L, N), jnp.int32),
            pltpu.VMEM((PPS, N), jnp.int32),
            pltpu.VMEM((PPS, L), jnp.int32),
            pltpu.VMEM((PPS, L), jnp.int32),
            pltpu.VMEM((PPS, L), jnp.float32),
            pltpu.VMEM((PPS * L,), jnp.int32),
            pltpu.VMEM((PPS * L,), jnp.int32),
            pltpu.VMEM((PPS, L), jnp.float32),
            pltpu.VMEM((PPS, L), jnp.float32),
            pltpu.SemaphoreType.DMA,
            pltpu.SemaphoreType.DMA,
        ],
    )
    return fn(gt, labt, d0t, d1t)


def _loss_body(den_ref, svs_ref, d0_ref, d1_ref, out_ref):
    den = den_ref[...]
    svs = svs_ref[...]
    d0 = d0_ref[...]
    d1 = d1_ref[...]
    pidx = lax.broadcasted_iota(jnp.int32, (P, L), 0)
    i = pidx // W
    j = pidx - i * W
    ni = i + d0
    nj = j + d1
    valid = (ni >= 0) & (ni < H) & (nj >= 0) & (nj < W)
    v = valid.astype(jnp.float32)
    count = jnp.sum(v, axis=1)
    tot = count * jnp.sum(jnp.log(den + 1e-6), axis=1) - jnp.sum(svs, axis=1)
    loss_p = jnp.where(count > 0, tot / (N * jnp.maximum(count, 1.0)), 0.0)
    out_ref[0, 0] = jnp.sum(loss_p) / P


def _loss(denom, svs, d0t, d1t):
    r = pl.pallas_call(
        _loss_body,
        in_specs=[
            pl.BlockSpec((P, N), lambda: (0, 0)),
            pl.BlockSpec((P, N), lambda: (0, 0)),
            pl.BlockSpec((P, L), lambda: (0, 0)),
            pl.BlockSpec((P, L), lambda: (0, 0)),
        ],
        out_specs=pl.BlockSpec(memory_space=pltpu.SMEM),
        out_shape=jax.ShapeDtypeStruct((1, 1), jnp.float32),
    )(denom, svs, d0t, d1t)
    return r[0, 0]


def kernel(features, labels, directions):
    feats = features.reshape(N, C, P)
    labt = labels.reshape(N, P).T
    d0t = directions[:, 0].reshape(N, P).T
    d1t = directions[:, 1].reshape(N, P).T
    g = _gram(feats)
    gt = jnp.transpose(g, (1, 2, 0)).reshape(P * P, N)
    denom, svs = _sc_gather(gt, labt, d0t, d1t)
    return _loss(denom, svs, d0t, d1t)


# speedup vs baseline: 23.6273x; 1.0855x over previous
import jax
import jax.numpy as jnp
from jax import lax
from jax.experimental import pallas as pl
from jax.experimental.pallas import tpu as pltpu
from jax.experimental.pallas import tpu_sc as plsc

INV_TEMP = 10.0
N, C, H, W = 16, 512, 16, 16
P = H * W
NC, NS, L = 1, 16, 16
NW = NC * NS
PPS = P // NW


GB = 4


def _gram_body(x_ref, g_ref):
    for bb in range(GB):
        x = x_ref[bb]
        n2 = jnp.sum(x * x, axis=0, keepdims=True)
        xn = x / jnp.maximum(jnp.sqrt(n2), 1e-12)
        g = lax.dot_general(
            xn, xn, (((0,), (0,)), ((), ())),
            preferred_element_type=jnp.float32,
        )
        g_ref[bb] = g * INV_TEMP


def _gram(feats):
    return pl.pallas_call(
        _gram_body,
        grid=(N // GB,),
        in_specs=[pl.BlockSpec((GB, C, P), lambda b: (b, 0, 0))],
        out_specs=pl.BlockSpec((GB, P, P), lambda b: (b, 0, 0)),
        out_shape=jax.ShapeDtypeStruct((N, P, P), jnp.float32),
    )(feats)


def _sc_body(gt_hbm, labt_hbm, d0_hbm, d1_hbm, denom_hbm, svs_hbm,
             rows_v, labrows_v, labc_v, d0v, d1v, vv, idxg, idxl,
             outd, outs, semg, seml):
    wid = lax.axis_index("s") * NC + lax.axis_index("c")
    base = wid * PPS
    pltpu.sync_copy(d0_hbm.at[pl.ds(base, PPS)], d0v)
    pltpu.sync_copy(d1_hbm.at[pl.ds(base, PPS)], d1v)
    pltpu.sync_copy(labt_hbm.at[pl.ds(base, PPS)], labc_v)
    for pi in range(PPS):
        p = base + pi
        i = p // W
        j = p - i * W
        ni = i + d0v[pi]
        nj = j + d1v[pi]
        valid = (ni >= 0) & (ni < H) & (nj >= 0) & (nj < W)
        q = jnp.clip(ni, 0, H - 1) * W + jnp.clip(nj, 0, W - 1)
        fz = jnp.zeros((L,), jnp.float32)
        vv[pi] = jnp.where(valid, fz + 1.0, fz)
        idxg[pl.ds(pi * L, L)] = p * P + q
        idxl[pl.ds(pi * L, L)] = q
    copies = []
    for ck in range(PPS * L // 128):
        d = pl.ds(ck * 128, 128)
        copies.append(pltpu.async_copy(gt_hbm.at[idxg.at[d]], rows_v.at[d], semg))
        copies.append(pltpu.async_copy(labt_hbm.at[idxl.at[d]], labrows_v.at[d], seml))
    for c in copies:
        c.wait()
    for pi in range(PPS):
        v = vv[pi]
        labc = labc_v[pi]
        fz = jnp.zeros((L,), jnp.float32)
        acc_d = fz
        acc_s = fz
        for n in range(N):
            r = pi * L + n
            nsplat = jnp.clip(d0v[pi] * 0 + n, 0, L - 1)
            vn = jnp.take(v, nsplat)
            s = rows_v[r]
            labn = labrows_v[r]
            mask = jnp.where(labn == labc, fz + 1.0, fz)
            acc_d = acc_d + jnp.exp(s) * mask * vn
            acc_s = acc_s + vn * s
        outd[pi] = acc_d
        outs[pi] = acc_s
    pltpu.sync_copy(outd, denom_hbm.at[pl.ds(base, PPS)])
    pltpu.sync_copy(outs, svs_hbm.at[pl.ds(base, PPS)])


def _sc_gather(gt, labt, d0t, d1t):
    fn = pl.kernel(
        _sc_body,
        out_type=[jax.ShapeDtypeStruct((P, N), jnp.float32),
                  jax.ShapeDtypeStruct((P, N), jnp.float32)],
        mesh=plsc.VectorSubcoreMesh(core_axis_name="c", subcore_axis_name="s",
                                    num_cores=NC, num_subcores=NS),
        compiler_params=pltpu.CompilerParams(use_tc_tiling_on_sc=False),
        scratch_types=[
            pltpu.VMEM((PPS * L, N), jnp.float32),
            pltpu.VMEM((PPS * L, N), jnp.int32),
            pltpu.VMEM((PPS, N), jnp.int32),
            pltpu.VMEM((PPS, L), jnp.int32),
            pltpu.VMEM((PPS, L), jnp.int32),
            pltpu.VMEM((PPS, L), jnp.float32),
            pltpu.VMEM((PPS * L,), jnp.int32),
            pltpu.VMEM((PPS * L,), jnp.int32),
            pltpu.VMEM((PPS, L), jnp.float32),
            pltpu.VMEM((PPS, L), jnp.float32),
            pltpu.SemaphoreType.DMA,
            pltpu.SemaphoreType.DMA,
        ],
    )
    return fn(gt, labt, d0t, d1t)


def _loss_body(den_ref, svs_ref, d0_ref, d1_ref, out_ref):
    den = den_ref[...]
    svs = svs_ref[...]
    d0 = d0_ref[...]
    d1 = d1_ref[...]
    pidx = lax.broadcasted_iota(jnp.int32, (P, L), 0)
    i = pidx // W
    j = pidx - i * W
    ni = i + d0
    nj = j + d1
    valid = (ni >= 0) & (ni < H) & (nj >= 0) & (nj < W)
    v = valid.astype(jnp.float32)
    count = jnp.sum(v, axis=1)
    tot = count * jnp.sum(jnp.log(den + 1e-6), axis=1) - jnp.sum(svs, axis=1)
    loss_p = jnp.where(count > 0, tot / (N * jnp.maximum(count, 1.0)), 0.0)
    out_ref[0, 0] = jnp.sum(loss_p) / P


def _loss(denom, svs, d0t, d1t):
    r = pl.pallas_call(
        _loss_body,
        in_specs=[
            pl.BlockSpec((P, N), lambda: (0, 0)),
            pl.BlockSpec((P, N), lambda: (0, 0)),
            pl.BlockSpec((P, L), lambda: (0, 0)),
            pl.BlockSpec((P, L), lambda: (0, 0)),
        ],
        out_specs=pl.BlockSpec(memory_space=pltpu.SMEM),
        out_shape=jax.ShapeDtypeStruct((1, 1), jnp.float32),
    )(denom, svs, d0t, d1t)
    return r[0, 0]


def kernel(features, labels, directions):
    feats = features.reshape(N, C, P)
    labt = labels.reshape(N, P).T
    d0t = directions[:, 0].reshape(N, P).T
    d1t = directions[:, 1].reshape(N, P).T
    g = _gram(feats)
    gt = jnp.transpose(g, (1, 2, 0)).reshape(P * P, N)
    denom, svs = _sc_gather(gt, labt, d0t, d1t)
    return _loss(denom, svs, d0t, d1t)


# speedup vs baseline: 23.7105x; 1.0035x over previous
import jax
import jax.numpy as jnp
from jax import lax
from jax.experimental import pallas as pl
from jax.experimental.pallas import tpu as pltpu
from jax.experimental.pallas import tpu_sc as plsc

INV_TEMP = 10.0
N, C, H, W = 16, 512, 16, 16
P = H * W
NC, NS, L = 1, 16, 16
NW = NC * NS
PPS = P // NW


GB = 8


def _gram_body(x_ref, g_ref):
    for bb in range(GB):
        x = x_ref[bb]
        n2 = jnp.sum(x * x, axis=0, keepdims=True)
        xn = x / jnp.maximum(jnp.sqrt(n2), 1e-12)
        g = lax.dot_general(
            xn, xn, (((0,), (0,)), ((), ())),
            preferred_element_type=jnp.float32,
        )
        g_ref[bb] = g * INV_TEMP


def _gram(feats):
    return pl.pallas_call(
        _gram_body,
        grid=(N // GB,),
        in_specs=[pl.BlockSpec((GB, C, P), lambda b: (b, 0, 0))],
        out_specs=pl.BlockSpec((GB, P, P), lambda b: (b, 0, 0)),
        out_shape=jax.ShapeDtypeStruct((N, P, P), jnp.float32),
    )(feats)


def _sc_body(gt_hbm, labt_hbm, d0_hbm, d1_hbm, denom_hbm, svs_hbm,
             rows_v, labrows_v, labc_v, d0v, d1v, vv, idxg, idxl,
             outd, outs, semg, seml):
    wid = lax.axis_index("s") * NC + lax.axis_index("c")
    base = wid * PPS
    pltpu.sync_copy(d0_hbm.at[pl.ds(base, PPS)], d0v)
    pltpu.sync_copy(d1_hbm.at[pl.ds(base, PPS)], d1v)
    pltpu.sync_copy(labt_hbm.at[pl.ds(base, PPS)], labc_v)
    for pi in range(PPS):
        p = base + pi
        i = p // W
        j = p - i * W
        ni = i + d0v[pi]
        nj = j + d1v[pi]
        valid = (ni >= 0) & (ni < H) & (nj >= 0) & (nj < W)
        q = jnp.clip(ni, 0, H - 1) * W + jnp.clip(nj, 0, W - 1)
        fz = jnp.zeros((L,), jnp.float32)
        vv[pi] = jnp.where(valid, fz + 1.0, fz)
        idxg[pl.ds(pi * L, L)] = p * P + q
        idxl[pl.ds(pi * L, L)] = q
    copies = []
    for ck in range(PPS * L // 128):
        d = pl.ds(ck * 128, 128)
        copies.append(pltpu.async_copy(gt_hbm.at[idxg.at[d]], rows_v.at[d], semg))
        copies.append(pltpu.async_copy(labt_hbm.at[idxl.at[d]], labrows_v.at[d], seml))
    for c in copies:
        c.wait()
    for pi in range(PPS):
        v = vv[pi]
        labc = labc_v[pi]
        fz = jnp.zeros((L,), jnp.float32)
        acc_d = fz
        acc_s = fz
        for n in range(N):
            r = pi * L + n
            nsplat = jnp.clip(d0v[pi] * 0 + n, 0, L - 1)
            vn = jnp.take(v, nsplat)
            s = rows_v[r]
            labn = labrows_v[r]
            mask = jnp.where(labn == labc, fz + 1.0, fz)
            acc_d = acc_d + jnp.exp(s) * mask * vn
            acc_s = acc_s + vn * s
        outd[pi] = acc_d
        outs[pi] = acc_s
    pltpu.sync_copy(outd, denom_hbm.at[pl.ds(base, PPS)])
    pltpu.sync_copy(outs, svs_hbm.at[pl.ds(base, PPS)])


def _sc_gather(gt, labt, d0t, d1t):
    fn = pl.kernel(
        _sc_body,
        out_type=[jax.ShapeDtypeStruct((P, N), jnp.float32),
                  jax.ShapeDtypeStruct((P, N), jnp.float32)],
        mesh=plsc.VectorSubcoreMesh(core_axis_name="c", subcore_axis_name="s",
                                    num_cores=NC, num_subcores=NS),
        compiler_params=pltpu.CompilerParams(use_tc_tiling_on_sc=False),
        scratch_types=[
            pltpu.VMEM((PPS * L, N), jnp.float32),
            pltpu.VMEM((PPS * L, N), jnp.int32),
            pltpu.VMEM((PPS, N), jnp.int32),
            pltpu.VMEM((PPS, L), jnp.int32),
            pltpu.VMEM((PPS, L), jnp.int32),
            pltpu.VMEM((PPS, L), jnp.float32),
            pltpu.VMEM((PPS * L,), jnp.int32),
            pltpu.VMEM((PPS * L,), jnp.int32),
            pltpu.VMEM((PPS, L), jnp.float32),
            pltpu.VMEM((PPS, L), jnp.float32),
            pltpu.SemaphoreType.DMA,
            pltpu.SemaphoreType.DMA,
        ],
    )
    return fn(gt, labt, d0t, d1t)


def _loss_body(den_ref, svs_ref, d0_ref, d1_ref, out_ref):
    den = den_ref[...]
    svs = svs_ref[...]
    d0 = d0_ref[...]
    d1 = d1_ref[...]
    pidx = lax.broadcasted_iota(jnp.int32, (P, L), 0)
    i = pidx // W
    j = pidx - i * W
    ni = i + d0
    nj = j + d1
    valid = (ni >= 0) & (ni < H) & (nj >= 0) & (nj < W)
    v = valid.astype(jnp.float32)
    count = jnp.sum(v, axis=1)
    tot = count * jnp.sum(jnp.log(den + 1e-6), axis=1) - jnp.sum(svs, axis=1)
    loss_p = jnp.where(count > 0, tot / (N * jnp.maximum(count, 1.0)), 0.0)
    out_ref[0, 0] = jnp.sum(loss_p) / P


def _loss(denom, svs, d0t, d1t):
    r = pl.pallas_call(
        _loss_body,
        in_specs=[
            pl.BlockSpec((P, N), lambda: (0, 0)),
            pl.BlockSpec((P, N), lambda: (0, 0)),
            pl.BlockSpec((P, L), lambda: (0, 0)),
            pl.BlockSpec((P, L), lambda: (0, 0)),
        ],
        out_specs=pl.BlockSpec(memory_space=pltpu.SMEM),
        out_shape=jax.ShapeDtypeStruct((1, 1), jnp.float32),
    )(denom, svs, d0t, d1t)
    return r[0, 0]


def kernel(features, labels, directions):
    feats = features.reshape(N, C, P)
    labt = labels.reshape(N, P).T
    d0t = directions[:, 0].reshape(N, P).T
    d1t = directions[:, 1].reshape(N, P).T
    g = _gram(feats)
    gt = jnp.transpose(g, (1, 2, 0)).reshape(P * P, N)
    denom, svs = _sc_gather(gt, labt, d0t, d1t)
    return _loss(denom, svs, d0t, d1t)


# speedup vs baseline: 24.7867x; 1.0454x over previous
import jax
import jax.numpy as jnp
from jax import lax
from jax.experimental import pallas as pl
from jax.experimental.pallas import tpu as pltpu
from jax.experimental.pallas import tpu_sc as plsc

INV_TEMP = 10.0
N, C, H, W = 16, 512, 16, 16
P = H * W
NC, NS, L = 1, 16, 16
NW = NC * NS
PPS = P // NW


GB = 8


def _gram_body(x_ref, g_ref):
    for bb in range(GB):
        x = x_ref[bb]
        n2 = jnp.sum(x * x, axis=0, keepdims=True)
        xn = x / jnp.maximum(jnp.sqrt(n2), 1e-12)
        g = lax.dot_general(
            xn, xn, (((0,), (0,)), ((), ())),
            preferred_element_type=jnp.float32,
        )
        g_ref[bb] = g * INV_TEMP


def _gram(feats):
    return pl.pallas_call(
        _gram_body,
        grid=(N // GB,),
        in_specs=[pl.BlockSpec((GB, C, P), lambda b: (b, 0, 0))],
        out_specs=pl.BlockSpec((GB, P, P), lambda b: (b, 0, 0)),
        out_shape=jax.ShapeDtypeStruct((N, P, P), jnp.float32),
    )(feats)


def _sc_body(gt_hbm, aux_hbm, denom_hbm,
             rows_v, labrows_v, labc_v, d0v, d1v, vv, idxg, idxl,
             outd, outs, semg, seml):
    wid = lax.axis_index("s") * NC + lax.axis_index("c")
    base = wid * PPS
    pltpu.sync_copy(aux_hbm.at[pl.ds(P + base, PPS)], d0v)
    pltpu.sync_copy(aux_hbm.at[pl.ds(2 * P + base, PPS)], d1v)
    pltpu.sync_copy(aux_hbm.at[pl.ds(base, PPS)], labc_v)
    for pi in range(PPS):
        p = base + pi
        i = p // W
        j = p - i * W
        ni = i + d0v[pi]
        nj = j + d1v[pi]
        valid = (ni >= 0) & (ni < H) & (nj >= 0) & (nj < W)
        q = jnp.clip(ni, 0, H - 1) * W + jnp.clip(nj, 0, W - 1)
        fz = jnp.zeros((L,), jnp.float32)
        vv[pi] = jnp.where(valid, fz + 1.0, fz)
        idxg[pl.ds(pi * L, L)] = p * P + q
        idxl[pl.ds(pi * L, L)] = q
    copies = []
    for ck in range(PPS * L // 128):
        d = pl.ds(ck * 128, 128)
        copies.append(pltpu.async_copy(gt_hbm.at[idxg.at[d]], rows_v.at[d], semg))
        copies.append(pltpu.async_copy(aux_hbm.at[idxl.at[d]], labrows_v.at[d], seml))
    for c in copies:
        c.wait()
    for pi in range(PPS):
        v = vv[pi]
        labc = labc_v[pi]
        fz = jnp.zeros((L,), jnp.float32)
        acc_d = fz
        acc_s = fz
        for n in range(N):
            r = pi * L + n
            nsplat = jnp.clip(d0v[pi] * 0 + n, 0, L - 1)
            vn = jnp.take(v, nsplat)
            s = rows_v[r]
            labn = labrows_v[r]
            mask = jnp.where(labn == labc, fz + 1.0, fz)
            acc_d = acc_d + jnp.exp(s) * mask * vn
            acc_s = acc_s + vn * s
        outd[pi] = acc_d
        outs[pi] = acc_s
    pltpu.sync_copy(outd, denom_hbm.at[pl.ds(base, PPS)])
    pltpu.sync_copy(outs, denom_hbm.at[pl.ds(P + base, PPS)])


def _sc_gather(gt, labt, d0t, d1t):
    fn = pl.kernel(
        _sc_body,
        out_type=jax.ShapeDtypeStruct((2 * P, N), jnp.float32),
        mesh=plsc.VectorSubcoreMesh(core_axis_name="c", subcore_axis_name="s",
                                    num_cores=NC, num_subcores=NS),
        compiler_params=pltpu.CompilerParams(use_tc_tiling_on_sc=False),
        scratch_types=[
            pltpu.VMEM((PPS * L, N), jnp.float32),
            pltpu.VMEM((PPS * L, N), jnp.int32),
            pltpu.VMEM((PPS, N), jnp.int32),
            pltpu.VMEM((PPS, L), jnp.int32),
            pltpu.VMEM((PPS, L), jnp.int32),
            pltpu.VMEM((PPS, L), jnp.float32),
            pltpu.VMEM((PPS * L,), jnp.int32),
            pltpu.VMEM((PPS * L,), jnp.int32),
            pltpu.VMEM((PPS, L), jnp.float32),
            pltpu.VMEM((PPS, L), jnp.float32),
            pltpu.SemaphoreType.DMA,
            pltpu.SemaphoreType.DMA,
        ],
    )
    aux = jnp.concatenate([labt, d0t, d1t], axis=0)
    out = fn(gt, aux)
    return out[:P], out[P:]


def _loss_body(den_ref, svs_ref, d0_ref, d1_ref, out_ref):
    den = den_ref[...]
    svs = svs_ref[...]
    d0 = d0_ref[...]
    d1 = d1_ref[...]
    pidx = lax.broadcasted_iota(jnp.int32, (P, L), 0)
    i = pidx // W
    j = pidx - i * W
    ni = i + d0
    nj = j + d1
    valid = (ni >= 0) & (ni < H) & (nj >= 0) & (nj < W)
    v = valid.astype(jnp.float32)
    count = jnp.sum(v, axis=1)
    tot = count * jnp.sum(jnp.log(den + 1e-6), axis=1) - jnp.sum(svs, axis=1)
    loss_p = jnp.where(count > 0, tot / (N * jnp.maximum(count, 1.0)), 0.0)
    out_ref[0, 0] = jnp.sum(loss_p) / P


def _loss(denom, svs, d0t, d1t):
    r = pl.pallas_call(
        _loss_body,
        in_specs=[
            pl.BlockSpec((P, N), lambda: (0, 0)),
            pl.BlockSpec((P, N), lambda: (0, 0)),
            pl.BlockSpec((P, L), lambda: (0, 0)),
            pl.BlockSpec((P, L), lambda: (0, 0)),
        ],
        out_specs=pl.BlockSpec(memory_space=pltpu.SMEM),
        out_shape=jax.ShapeDtypeStruct((1, 1), jnp.float32),
    )(denom, svs, d0t, d1t)
    return r[0, 0]


def kernel(features, labels, directions):
    feats = features.reshape(N, C, P)
    labt = labels.reshape(N, P).T
    d0t = directions[:, 0].reshape(N, P).T
    d1t = directions[:, 1].reshape(N, P).T
    g = _gram(feats)
    gt = jnp.transpose(g, (1, 2, 0)).reshape(P * P, N)
    denom, svs = _sc_gather(gt, labt, d0t, d1t)
    return _loss(denom, svs, d0t, d1t)
